# baseline (speedup 1.0000x reference)
import functools

import jax
import jax.numpy as jnp
from jax import lax
from jax.experimental import pallas as pl
from jax.experimental.pallas import tpu as pltpu
from jax.experimental.pallas import tpu_sc as plsc

_NC = 2
_NS = 16
_LANES = 16
_GRP = 9


def _mm_body(h_ref, w_ref, o_ref):
    o_ref[...] = jnp.dot(h_ref[...], w_ref[...],
                         preferred_element_type=jnp.float32)


def _mid_body(p_ref, b_ref, o_ref):
    n = o_ref.shape[0]
    o_ref[...] = jnp.maximum(p_ref[0, :n] + p_ref[1, :n] + b_ref[...], 0.0)


def _fin_body(p_ref, w_ref, b_ref, o_ref):
    n = o_ref.shape[0]
    s2 = p_ref[0, :n] + p_ref[1, :n]
    logits = jnp.dot(s2, w_ref[...], preferred_element_type=jnp.float32)
    logits = jnp.maximum(logits + b_ref[...], 0.0)
    m = jnp.max(logits, axis=1, keepdims=True)
    x = logits - m
    lse = jnp.log(jnp.sum(jnp.exp(x), axis=1, keepdims=True))
    o_ref[...] = x - lse


@functools.lru_cache(maxsize=None)
def _make_spmm(n_rows: int, width: int, n_chunks: int):
    rpt = (n_rows + _NS - 1) // _NS
    rows_per_tile = (rpt + 7) // 8 * 8
    n_pad = rows_per_tile * _NS
    grp = _GRP
    n_steps = n_chunks // grp
    assert n_steps * grp == n_chunks
    mesh = plsc.VectorSubcoreMesh(core_axis_name="c", subcore_axis_name="s")

    @functools.partial(
        pl.kernel,
        out_type=jax.ShapeDtypeStruct((_NC, n_pad, width), jnp.float32),
        mesh=mesh,
        scratch_types=[
            pltpu.VMEM((n_chunks, 128), jnp.int32),
            pltpu.VMEM((n_chunks, 128), jnp.int32),
            pltpu.VMEM((n_chunks, 128), jnp.float32),
            pltpu.VMEM((2, grp * 128, width), jnp.float32),
            pltpu.VMEM((rows_per_tile, width), jnp.float32),
            pltpu.VMEM_SHARED((n_pad, width), jnp.float32),
            pltpu.SemaphoreType.DMA,
        ],
        compiler_params=pltpu.CompilerParams(use_tc_tiling_on_sc=False),
    )
    def spmm(mat_hbm, col_hbm, row_hbm, val_hbm, out_hbm,
             colv, rowv, valv, rbuf, zbuf, acc, sem):
        c = lax.axis_index("c")
        tid = lax.axis_index("s")
        w = c * _NS + tid

        pltpu.sync_copy(col_hbm.at[w], colv)
        pltpu.sync_copy(row_hbm.at[w], rowv)
        pltpu.sync_copy(val_hbm.at[w], valv)

        def zbody(i, carry):
            zbuf[i, :] = jnp.zeros((width,), jnp.float32)
            return carry
        lax.fori_loop(0, rows_per_tile, zbody, 0)
        pltpu.sync_copy(zbuf, acc.at[pl.ds(tid * rows_per_tile, rows_per_tile)])
        plsc.subcore_barrier()

        for i in range(grp):
            pltpu.async_copy(mat_hbm.at[colv.at[i]],
                             rbuf.at[0, pl.ds(i * 128, 128)], sem)

        def superstep(s, carry):
            p = lax.rem(s, 2)
            pltpu.make_async_copy(mat_hbm.at[pl.ds(0, grp * 128)],
                                  rbuf.at[p], sem).wait()

            @pl.when(s < n_steps - 1)
            def _():
                base = jnp.minimum(s + 1, n_steps - 1) * grp
                for i in range(grp):
                    pltpu.async_copy(mat_hbm.at[colv.at[base + i]],
                                     rbuf.at[1 - p, pl.ds(i * 128, 128)], sem)

            for i in range(grp):
                cidx = s * grp + i

                def scale_g(g, carry2):
                    vals = valv[cidx, pl.ds(g * _LANES, _LANES)]
                    for t in range(_LANES):
                        e = i * 128 + g * _LANES + t
                        bv = vals.at[jnp.full((_LANES,), t, jnp.int32)].get(
                            mode="promise_in_bounds")
                        rbuf[p, e, :] = rbuf[p, e, :] * bv
                    return carry2
                lax.fori_loop(0, 8, scale_g, 0)
                pltpu.sync_copy(rbuf.at[p, pl.ds(i * 128, 128)],
                                acc.at[rowv.at[cidx]], add=True)
            return carry
        lax.fori_loop(0, n_steps, superstep, 0)
        plsc.subcore_barrier()

        pltpu.sync_copy(acc.at[pl.ds(tid * rows_per_tile, rows_per_tile)],
                        out_hbm.at[c, pl.ds(tid * rows_per_tile, rows_per_tile)])

    return spmm


def kernel(H, adj_row, adj_col, adj_val, W1, b1, W2, b2):
    n, d = H.shape
    hid = W1.shape[1]
    ncls = W2.shape[1]
    e_tot = adj_row.shape[0]

    n_chunks = -(-e_tot // (_NC * _NS * 128 * _GRP)) * _GRP
    e_pad = _NC * _NS * 128 * n_chunks
    pad = e_pad - e_tot
    pad_idx = jnp.arange(pad, dtype=jnp.int32) % n
    col3 = jnp.concatenate([adj_col, pad_idx]).reshape(_NC * _NS, n_chunks, 128)
    row3 = jnp.concatenate([adj_row, pad_idx]).reshape(_NC * _NS, n_chunks, 128)
    val3 = jnp.concatenate(
        [adj_val, jnp.zeros((pad,), jnp.float32)]).reshape(_NC * _NS, n_chunks, 128)

    spmm = _make_spmm(n, hid, n_chunks)

    hw1 = pl.pallas_call(
        _mm_body,
        out_shape=jax.ShapeDtypeStruct((n, hid), jnp.float32),
    )(H, W1)

    p1 = spmm(hw1, col3, row3, val3)

    h1 = pl.pallas_call(
        _mid_body,
        out_shape=jax.ShapeDtypeStruct((n, hid), jnp.float32),
    )(p1, b1.reshape(1, hid))

    p2 = spmm(h1, col3, row3, val3)

    out = pl.pallas_call(
        _fin_body,
        out_shape=jax.ShapeDtypeStruct((n, ncls), jnp.float32),
    )(p2, W2, b2.reshape(1, ncls))

    return out


# speedup vs baseline: 1.1200x; 1.1200x over previous
import functools

import jax
import jax.numpy as jnp
from jax import lax
from jax.experimental import pallas as pl
from jax.experimental.pallas import tpu as pltpu
from jax.experimental.pallas import tpu_sc as plsc

_NC = 2
_NS = 16
_LANES = 16
_GRP = 9


def _mm_body(h_ref, w_ref, o_ref):
    o_ref[...] = jnp.dot(h_ref[...], w_ref[...],
                         preferred_element_type=jnp.float32)


def _mid_body(p_ref, b_ref, o_ref):
    half = o_ref.shape[0]
    o_ref[...] = jnp.maximum(
        p_ref[:half] + p_ref[half:] + b_ref[...], 0.0)


def _fin_body(p_ref, w_ref, b_ref, o_ref):
    n = o_ref.shape[0]
    s2 = p_ref[0, :n] + p_ref[1, :n]
    logits = jnp.dot(s2, w_ref[...], preferred_element_type=jnp.float32)
    logits = jnp.maximum(logits + b_ref[...], 0.0)
    m = jnp.max(logits, axis=1, keepdims=True)
    x = logits - m
    lse = jnp.log(jnp.sum(jnp.exp(x), axis=1, keepdims=True))
    o_ref[...] = x - lse


@functools.lru_cache(maxsize=None)
def _make_spmm(n_rows: int, width: int, n_chunks: int):
    rpt = (n_rows + _NS - 1) // _NS
    rows_per_tile = (rpt + 7) // 8 * 8
    n_pad = rows_per_tile * _NS
    grp = _GRP
    n_steps = n_chunks // grp
    assert n_steps * grp == n_chunks
    mesh = plsc.VectorSubcoreMesh(core_axis_name="c", subcore_axis_name="s")

    @functools.partial(
        pl.kernel,
        out_type=jax.ShapeDtypeStruct((_NC, n_pad, width), jnp.float32),
        mesh=mesh,
        scratch_types=[
            pltpu.VMEM((n_chunks, 128), jnp.int32),
            pltpu.VMEM((n_chunks, 128), jnp.int32),
            pltpu.VMEM((n_chunks, 128), jnp.float32),
            pltpu.VMEM((2, grp * 128, width), jnp.float32),
            pltpu.VMEM((rows_per_tile, width), jnp.float32),
            pltpu.VMEM_SHARED((n_pad, width), jnp.float32),
            pltpu.SemaphoreType.DMA,
        ],
        compiler_params=pltpu.CompilerParams(use_tc_tiling_on_sc=False),
    )
    def spmm(mat_hbm, col_hbm, row_hbm, val_hbm, out_hbm,
             colv, rowv, valv, rbuf, zbuf, acc, sem):
        c = lax.axis_index("c")
        tid = lax.axis_index("s")
        w = c * _NS + tid

        pltpu.sync_copy(col_hbm.at[w], colv)
        pltpu.sync_copy(row_hbm.at[w], rowv)
        pltpu.sync_copy(val_hbm.at[w], valv)

        def zbody(i, carry):
            zbuf[i, :] = jnp.zeros((width,), jnp.float32)
            return carry
        lax.fori_loop(0, rows_per_tile, zbody, 0)
        pltpu.sync_copy(zbuf, acc.at[pl.ds(tid * rows_per_tile, rows_per_tile)])
        plsc.subcore_barrier()

        for i in range(grp):
            pltpu.async_copy(mat_hbm.at[colv.at[i]],
                             rbuf.at[0, pl.ds(i * 128, 128)], sem)

        def superstep(s, carry):
            p = lax.rem(s, 2)
            pltpu.make_async_copy(mat_hbm.at[pl.ds(0, grp * 128)],
                                  rbuf.at[p], sem).wait()

            @pl.when(s < n_steps - 1)
            def _():
                base = jnp.minimum(s + 1, n_steps - 1) * grp
                for i in range(grp):
                    pltpu.async_copy(mat_hbm.at[colv.at[base + i]],
                                     rbuf.at[1 - p, pl.ds(i * 128, 128)], sem)

            for i in range(grp):
                cidx = s * grp + i

                def scale_g(g, carry2):
                    vals = valv[cidx, pl.ds(g * _LANES, _LANES)]
                    for t in range(_LANES):
                        e = i * 128 + g * _LANES + t
                        bv = vals.at[jnp.full((_LANES,), t, jnp.int32)].get(
                            mode="promise_in_bounds")
                        rbuf[p, e, :] = rbuf[p, e, :] * bv
                    return carry2
                lax.fori_loop(0, 8, scale_g, 0)
                pltpu.sync_copy(rbuf.at[p, pl.ds(i * 128, 128)],
                                acc.at[rowv.at[cidx]], add=True)
            return carry
        lax.fori_loop(0, n_steps, superstep, 0)
        plsc.subcore_barrier()

        pltpu.sync_copy(acc.at[pl.ds(tid * rows_per_tile, rows_per_tile)],
                        out_hbm.at[c, pl.ds(tid * rows_per_tile, rows_per_tile)])

    return spmm


def kernel(H, adj_row, adj_col, adj_val, W1, b1, W2, b2):
    n, d = H.shape
    hid = W1.shape[1]
    ncls = W2.shape[1]
    e_tot = adj_row.shape[0]

    n_chunks = -(-e_tot // (_NC * _NS * 128 * _GRP)) * _GRP
    e_pad = _NC * _NS * 128 * n_chunks
    pad = e_pad - e_tot
    pad_idx = jnp.arange(pad, dtype=jnp.int32) % n
    col3 = jnp.concatenate([adj_col, pad_idx]).reshape(_NC * _NS, n_chunks, 128)
    row3 = jnp.concatenate([adj_row, pad_idx]).reshape(_NC * _NS, n_chunks, 128)
    val3 = jnp.concatenate(
        [adj_val, jnp.zeros((pad,), jnp.float32)]).reshape(_NC * _NS, n_chunks, 128)

    spmm = _make_spmm(n, hid, n_chunks)

    hw1 = pl.pallas_call(
        _mm_body,
        out_shape=jax.ShapeDtypeStruct((n, hid), jnp.float32),
    )(H, W1)

    p1 = spmm(hw1, col3, row3, val3)

    n_pad = p1.shape[1]
    packed_rows = _NC * n_pad * hid // 128
    h1_2d = pl.pallas_call(
        _mid_body,
        out_shape=jax.ShapeDtypeStruct((packed_rows // 2, 128), jnp.float32),
    )(p1.reshape(packed_rows, 128), jnp.tile(b1, 128 // hid).reshape(1, 128))

    p2 = spmm(h1_2d.reshape(n_pad, hid), col3, row3, val3)

    out = pl.pallas_call(
        _fin_body,
        out_shape=jax.ShapeDtypeStruct((n, ncls), jnp.float32),
    )(p2, W2, b2.reshape(1, ncls))

    return out


# speedup vs baseline: 1.1534x; 1.0298x over previous
import functools

import jax
import jax.numpy as jnp
from jax import lax
from jax.experimental import pallas as pl
from jax.experimental.pallas import tpu as pltpu
from jax.experimental.pallas import tpu_sc as plsc

_NC = 2
_NS = 16
_LANES = 16
_GRP = 9


def _mm_body(h_ref, w_ref, o_ref):
    hw = jnp.dot(h_ref[...], w_ref[...], preferred_element_type=jnp.float32)
    o_ref[...] = jnp.concatenate(
        [hw, jnp.zeros((hw.shape[0], 128 - hw.shape[1]), jnp.float32)], axis=1)


def _mid_body(p_ref, b_ref, o_ref):
    half = o_ref.shape[0]
    o_ref[...] = jnp.maximum(
        p_ref[:half] + p_ref[half:] + b_ref[...], 0.0)


def _fin_body(p_ref, w_ref, b_ref, o_ref):
    n = o_ref.shape[0]
    s2 = p_ref[0, :n] + p_ref[1, :n]
    logits = jnp.dot(s2, w_ref[...], preferred_element_type=jnp.float32)
    logits = jnp.maximum(logits + b_ref[...], 0.0)
    m = jnp.max(logits, axis=1, keepdims=True)
    x = logits - m
    lse = jnp.log(jnp.sum(jnp.exp(x), axis=1, keepdims=True))
    o_ref[...] = x - lse


@functools.lru_cache(maxsize=None)
def _make_spmm(n_rows: int, width: int, n_chunks: int, idx_shift: int = 0):
    rpt = (n_rows + _NS - 1) // _NS
    rows_per_tile = (rpt + 7) // 8 * 8
    n_pad = rows_per_tile * _NS
    grp = _GRP
    n_steps = n_chunks // grp
    assert n_steps * grp == n_chunks
    mesh = plsc.VectorSubcoreMesh(core_axis_name="c", subcore_axis_name="s")

    @functools.partial(
        pl.kernel,
        out_type=jax.ShapeDtypeStruct((_NC, n_pad, width), jnp.float32),
        mesh=mesh,
        scratch_types=[
            pltpu.VMEM((n_chunks, 128), jnp.int32),
            pltpu.VMEM((n_chunks, 128), jnp.int32),
            pltpu.VMEM((n_chunks, 128), jnp.float32),
            pltpu.VMEM((2, grp * 128, width), jnp.float32),
            pltpu.VMEM((rows_per_tile, width), jnp.float32),
            pltpu.VMEM_SHARED((n_pad, width), jnp.float32),
            pltpu.SemaphoreType.DMA,
        ],
        compiler_params=pltpu.CompilerParams(use_tc_tiling_on_sc=False),
    )
    def spmm(mat_hbm, col_hbm, row_hbm, val_hbm, out_hbm,
             colv, rowv, valv, rbuf, zbuf, acc, sem):
        c = lax.axis_index("c")
        tid = lax.axis_index("s")
        w = c * _NS + tid

        pltpu.sync_copy(col_hbm.at[w], colv)
        pltpu.sync_copy(row_hbm.at[w], rowv)
        pltpu.sync_copy(val_hbm.at[w], valv)

        if idx_shift:
            def shift_body(j, carry):
                for g in range(8):
                    sl = pl.ds(g * _LANES, _LANES)
                    colv[j, sl] = colv[j, sl] * jnp.int32(1 << idx_shift)
                return carry
            lax.fori_loop(0, n_chunks, shift_body, 0)

        def zbody(i, carry):
            zbuf[i, :] = jnp.zeros((width,), jnp.float32)
            return carry
        lax.fori_loop(0, rows_per_tile, zbody, 0)
        pltpu.sync_copy(zbuf, acc.at[pl.ds(tid * rows_per_tile, rows_per_tile)])
        plsc.subcore_barrier()

        for i in range(grp):
            pltpu.async_copy(mat_hbm.at[colv.at[i]],
                             rbuf.at[0, pl.ds(i * 128, 128)], sem)

        def superstep(s, carry):
            p = lax.rem(s, 2)
            pltpu.make_async_copy(mat_hbm.at[pl.ds(0, grp * 128)],
                                  rbuf.at[p], sem).wait()

            @pl.when(s < n_steps - 1)
            def _():
                base = jnp.minimum(s + 1, n_steps - 1) * grp
                for i in range(grp):
                    pltpu.async_copy(mat_hbm.at[colv.at[base + i]],
                                     rbuf.at[1 - p, pl.ds(i * 128, 128)], sem)

            for i in range(grp):
                cidx = s * grp + i

                def scale_g(g, carry2):
                    vals = valv[cidx, pl.ds(g * _LANES, _LANES)]
                    for t in range(_LANES):
                        e = i * 128 + g * _LANES + t
                        bv = vals.at[jnp.full((_LANES,), t, jnp.int32)].get(
                            mode="promise_in_bounds")
                        rbuf[p, e, :] = rbuf[p, e, :] * bv
                    return carry2
                lax.fori_loop(0, 8, scale_g, 0)
                pltpu.sync_copy(rbuf.at[p, pl.ds(i * 128, 128)],
                                acc.at[rowv.at[cidx]], add=True)
            return carry
        lax.fori_loop(0, n_steps, superstep, 0)
        plsc.subcore_barrier()

        pltpu.sync_copy(acc.at[pl.ds(tid * rows_per_tile, rows_per_tile)],
                        out_hbm.at[c, pl.ds(tid * rows_per_tile, rows_per_tile)])

    return spmm


def kernel(H, adj_row, adj_col, adj_val, W1, b1, W2, b2):
    n, d = H.shape
    hid = W1.shape[1]
    ncls = W2.shape[1]
    e_tot = adj_row.shape[0]

    n_chunks = -(-e_tot // (_NC * _NS * 128 * _GRP)) * _GRP
    e_pad = _NC * _NS * 128 * n_chunks
    pad = e_pad - e_tot
    pad_idx = jnp.arange(pad, dtype=jnp.int32) % n
    col3 = jnp.concatenate([adj_col, pad_idx]).reshape(_NC * _NS, n_chunks, 128)
    row3 = jnp.concatenate([adj_row, pad_idx]).reshape(_NC * _NS, n_chunks, 128)
    val3 = jnp.concatenate(
        [adj_val, jnp.zeros((pad,), jnp.float32)]).reshape(_NC * _NS, n_chunks, 128)

    spmm1 = _make_spmm(n, hid, n_chunks, idx_shift=3)
    spmm = _make_spmm(n, hid, n_chunks)

    hw1_wide = pl.pallas_call(
        _mm_body,
        out_shape=jax.ShapeDtypeStruct((n, 128), jnp.float32),
    )(H, W1)

    p1 = spmm1(hw1_wide.reshape(8 * n, hid), col3, row3, val3)

    n_pad = p1.shape[1]
    packed_rows = _NC * n_pad * hid // 128
    h1_2d = pl.pallas_call(
        _mid_body,
        out_shape=jax.ShapeDtypeStruct((packed_rows // 2, 128), jnp.float32),
    )(p1.reshape(packed_rows, 128), jnp.tile(b1, 128 // hid).reshape(1, 128))

    p2 = spmm(h1_2d.reshape(n_pad, hid), col3, row3, val3)

    out = pl.pallas_call(
        _fin_body,
        out_shape=jax.ShapeDtypeStruct((n, ncls), jnp.float32),
    )(p2, W2, b2.reshape(1, ncls))

    return out


# speedup vs baseline: 1.1962x; 1.0371x over previous
import functools

import jax
import jax.numpy as jnp
from jax import lax
from jax.experimental import pallas as pl
from jax.experimental.pallas import tpu as pltpu
from jax.experimental.pallas import tpu_sc as plsc

_NC = 2
_NS = 16
_LANES = 16
_GRP = 9


def _mm_body(h_ref, w_ref, o_ref):
    hw = jnp.dot(h_ref[...], w_ref[...], preferred_element_type=jnp.float32)
    o_ref[...] = jnp.concatenate(
        [hw, jnp.zeros((hw.shape[0], 128 - hw.shape[1]), jnp.float32)], axis=1)


def _mid_body(p_ref, b_ref, o_ref):
    half = o_ref.shape[0]
    o_ref[...] = jnp.maximum(
        p_ref[:half] + p_ref[half:] + b_ref[...], 0.0)


def _fin_body(p_ref, w_ref, b_ref, o_ref):
    half = p_ref.shape[0] // 2
    ncls = o_ref.shape[1] // 8
    s2p = p_ref[:half] + p_ref[half:]
    z = jnp.dot(s2p, w_ref[...], preferred_element_type=jnp.float32)
    z = jnp.maximum(z + b_ref[...], 0.0)
    pieces = []
    for g in range(8):
        zg = z[:, g * ncls:(g + 1) * ncls]
        m = jnp.max(zg, axis=1, keepdims=True)
        x = zg - m
        lse = jnp.log(jnp.sum(jnp.exp(x), axis=1, keepdims=True))
        pieces.append(x - lse)
    o_ref[...] = jnp.concatenate(pieces, axis=1)


@functools.lru_cache(maxsize=None)
def _make_spmm(n_rows: int, width: int, n_chunks: int, idx_shift: int = 0):
    rpt = (n_rows + _NS - 1) // _NS
    rows_per_tile = (rpt + 7) // 8 * 8
    n_pad = rows_per_tile * _NS
    grp = _GRP
    n_steps = n_chunks // grp
    assert n_steps * grp == n_chunks
    mesh = plsc.VectorSubcoreMesh(core_axis_name="c", subcore_axis_name="s")

    @functools.partial(
        pl.kernel,
        out_type=jax.ShapeDtypeStruct((_NC, n_pad, width), jnp.float32),
        mesh=mesh,
        scratch_types=[
            pltpu.VMEM((n_chunks, 128), jnp.int32),
            pltpu.VMEM((n_chunks, 128), jnp.int32),
            pltpu.VMEM((n_chunks, 128), jnp.float32),
            pltpu.VMEM((2, grp * 128, width), jnp.float32),
            pltpu.VMEM((rows_per_tile, width), jnp.float32),
            pltpu.VMEM_SHARED((n_pad, width), jnp.float32),
            pltpu.SemaphoreType.DMA,
        ],
        compiler_params=pltpu.CompilerParams(use_tc_tiling_on_sc=False),
    )
    def spmm(mat_hbm, col_hbm, row_hbm, val_hbm, out_hbm,
             colv, rowv, valv, rbuf, zbuf, acc, sem):
        c = lax.axis_index("c")
        tid = lax.axis_index("s")
        w = c * _NS + tid

        pltpu.sync_copy(col_hbm.at[w], colv)
        pltpu.sync_copy(row_hbm.at[w], rowv)
        pltpu.sync_copy(val_hbm.at[w], valv)

        if idx_shift:
            def shift_body(j, carry):
                for g in range(8):
                    sl = pl.ds(g * _LANES, _LANES)
                    colv[j, sl] = colv[j, sl] * jnp.int32(1 << idx_shift)
                return carry
            lax.fori_loop(0, n_chunks, shift_body, 0)

        def zbody(i, carry):
            zbuf[i, :] = jnp.zeros((width,), jnp.float32)
            return carry
        lax.fori_loop(0, rows_per_tile, zbody, 0)
        pltpu.sync_copy(zbuf, acc.at[pl.ds(tid * rows_per_tile, rows_per_tile)])
        plsc.subcore_barrier()

        for i in range(grp):
            pltpu.async_copy(mat_hbm.at[colv.at[i]],
                             rbuf.at[0, pl.ds(i * 128, 128)], sem)

        def superstep(s, carry):
            p = lax.rem(s, 2)
            pltpu.make_async_copy(mat_hbm.at[pl.ds(0, grp * 128)],
                                  rbuf.at[p], sem).wait()

            @pl.when(s < n_steps - 1)
            def _():
                base = jnp.minimum(s + 1, n_steps - 1) * grp
                for i in range(grp):
                    pltpu.async_copy(mat_hbm.at[colv.at[base + i]],
                                     rbuf.at[1 - p, pl.ds(i * 128, 128)], sem)

            for i in range(grp):
                cidx = s * grp + i

                def scale_g(g, carry2):
                    vals = valv[cidx, pl.ds(g * _LANES, _LANES)]
                    for t in range(_LANES):
                        e = i * 128 + g * _LANES + t
                        bv = vals.at[jnp.full((_LANES,), t, jnp.int32)].get(
                            mode="promise_in_bounds")
                        rbuf[p, e, :] = rbuf[p, e, :] * bv
                    return carry2
                lax.fori_loop(0, 8, scale_g, 0)
                pltpu.sync_copy(rbuf.at[p, pl.ds(i * 128, 128)],
                                acc.at[rowv.at[cidx]], add=True)
            return carry
        lax.fori_loop(0, n_steps, superstep, 0)
        plsc.subcore_barrier()

        pltpu.sync_copy(acc.at[pl.ds(tid * rows_per_tile, rows_per_tile)],
                        out_hbm.at[c, pl.ds(tid * rows_per_tile, rows_per_tile)])

    return spmm


def kernel(H, adj_row, adj_col, adj_val, W1, b1, W2, b2):
    n, d = H.shape
    hid = W1.shape[1]
    ncls = W2.shape[1]
    e_tot = adj_row.shape[0]

    n_chunks = -(-e_tot // (_NC * _NS * 128 * _GRP)) * _GRP
    e_pad = _NC * _NS * 128 * n_chunks
    pad = e_pad - e_tot
    pad_idx = jnp.arange(pad, dtype=jnp.int32) % n
    col3 = jnp.concatenate([adj_col, pad_idx]).reshape(_NC * _NS, n_chunks, 128)
    row3 = jnp.concatenate([adj_row, pad_idx]).reshape(_NC * _NS, n_chunks, 128)
    val3 = jnp.concatenate(
        [adj_val, jnp.zeros((pad,), jnp.float32)]).reshape(_NC * _NS, n_chunks, 128)

    spmm1 = _make_spmm(n, hid, n_chunks, idx_shift=3)
    spmm = _make_spmm(n, hid, n_chunks)

    hw1_wide = pl.pallas_call(
        _mm_body,
        out_shape=jax.ShapeDtypeStruct((n, 128), jnp.float32),
    )(H, W1)

    p1 = spmm1(hw1_wide.reshape(8 * n, hid), col3, row3, val3)

    n_pad = p1.shape[1]
    packed_rows = _NC * n_pad * hid // 128
    h1_2d = pl.pallas_call(
        _mid_body,
        out_shape=jax.ShapeDtypeStruct((packed_rows // 2, 128), jnp.float32),
    )(p1.reshape(packed_rows, 128), jnp.tile(b1, 128 // hid).reshape(1, 128))

    p2 = spmm(h1_2d.reshape(n_pad, hid), col3, row3, val3)

    w2big = jnp.kron(jnp.eye(8, dtype=jnp.float32), W2)
    b2big = jnp.tile(b2, 8).reshape(1, 8 * ncls)
    fin_p = pl.pallas_call(
        _fin_body,
        out_shape=jax.ShapeDtypeStruct((packed_rows // 2, 8 * ncls),
                                       jnp.float32),
    )(p2.reshape(packed_rows, 128), w2big, b2big)

    return fin_p.reshape(n_pad, ncls)[:n]


# speedup vs baseline: 1.5091x; 1.2615x over previous
import functools

import jax
import jax.numpy as jnp
from jax import lax
from jax.experimental import pallas as pl
from jax.experimental.pallas import tpu as pltpu
from jax.experimental.pallas import tpu_sc as plsc

_NC = 2
_NS = 16
_LANES = 16
_GRP = 9


def _mm_body(h_ref, w_ref, o_ref):
    hw = jnp.dot(h_ref[...], w_ref[...], preferred_element_type=jnp.float32)
    o_ref[...] = jnp.concatenate(
        [hw, jnp.zeros((hw.shape[0], 128 - hw.shape[1]), jnp.float32)], axis=1)


def _mid_body(p_ref, b_ref, o_ref):
    half = o_ref.shape[0]
    o_ref[...] = jnp.maximum(
        p_ref[:half] + p_ref[half:] + b_ref[...], 0.0)


def _fin_body(p_ref, w_ref, b_ref, o_ref):
    half = p_ref.shape[0] // 2
    ncls = o_ref.shape[1] // 8
    s2p = p_ref[:half] + p_ref[half:]
    z = jnp.dot(s2p, w_ref[...], preferred_element_type=jnp.float32)
    z = jnp.maximum(z + b_ref[...], 0.0)
    pieces = []
    for g in range(8):
        zg = z[:, g * ncls:(g + 1) * ncls]
        m = jnp.max(zg, axis=1, keepdims=True)
        x = zg - m
        lse = jnp.log(jnp.sum(jnp.exp(x), axis=1, keepdims=True))
        pieces.append(x - lse)
    o_ref[...] = jnp.concatenate(pieces, axis=1)


@functools.lru_cache(maxsize=None)
def _make_spmm(n_rows: int, width: int, n_chunks: int, idx_shift: int = 0):
    rpt = (n_rows + _NS - 1) // _NS
    rows_per_tile = (rpt + 7) // 8 * 8
    n_pad = rows_per_tile * _NS
    grp = _GRP
    n_steps = n_chunks // grp
    assert n_steps * grp == n_chunks
    mesh = plsc.VectorSubcoreMesh(core_axis_name="c", subcore_axis_name="s")

    @functools.partial(
        pl.kernel,
        out_type=jax.ShapeDtypeStruct((_NC, n_pad, width), jnp.float32),
        mesh=mesh,
        scratch_types=[
            pltpu.VMEM((n_chunks, 128), jnp.int32),
            pltpu.VMEM((n_chunks, 128), jnp.int32),
            pltpu.VMEM((n_chunks, 128), jnp.float32),
            pltpu.VMEM((2, grp * 128, width), jnp.float32),
            pltpu.VMEM((rows_per_tile, width), jnp.float32),
            pltpu.VMEM_SHARED((n_pad, width), jnp.float32),
            pltpu.SemaphoreType.DMA,
            pltpu.SemaphoreType.DMA,
        ],
        compiler_params=pltpu.CompilerParams(use_tc_tiling_on_sc=False),
    )
    def spmm(mat_hbm, col_hbm, row_hbm, val_hbm, out_hbm,
             colv, rowv, valv, rbuf, zbuf, acc, gsem, ssem):
        c = lax.axis_index("c")
        tid = lax.axis_index("s")
        w = c * _NS + tid

        pltpu.sync_copy(col_hbm.at[w], colv)
        d_row = pltpu.async_copy(row_hbm.at[w], rowv, ssem)
        d_val = pltpu.async_copy(val_hbm.at[w], valv, ssem)

        if idx_shift:
            def shift_body(j, carry):
                for g in range(8):
                    sl = pl.ds(g * _LANES, _LANES)
                    colv[j, sl] = colv[j, sl] * jnp.int32(1 << idx_shift)
                return carry
            lax.fori_loop(0, n_chunks, shift_body, 0)

        for i in range(grp):
            pltpu.async_copy(mat_hbm.at[colv.at[i]],
                             rbuf.at[0, pl.ds(i * 128, 128)], gsem)

        def zbody(i, carry):
            for u in range(4):
                zbuf[i * 4 + u, :] = jnp.zeros((width,), jnp.float32)
            return carry
        lax.fori_loop(0, rows_per_tile // 4, zbody, 0)
        pltpu.sync_copy(zbuf, acc.at[pl.ds(tid * rows_per_tile, rows_per_tile)])
        d_row.wait()
        d_val.wait()
        plsc.subcore_barrier()

        def superstep(s, carry):
            p = lax.rem(s, 2)
            pltpu.make_async_copy(mat_hbm.at[pl.ds(0, grp * 128)],
                                  rbuf.at[p], gsem).wait()

            @pl.when(s > 0)
            def _():
                pltpu.make_async_copy(rbuf.at[1 - p],
                                      acc.at[pl.ds(0, grp * 128)],
                                      ssem).wait()

            @pl.when(s < n_steps - 1)
            def _():
                base = jnp.minimum(s + 1, n_steps - 1) * grp
                for i in range(grp):
                    pltpu.async_copy(mat_hbm.at[colv.at[base + i]],
                                     rbuf.at[1 - p, pl.ds(i * 128, 128)], gsem)

            for i in range(grp):
                cidx = s * grp + i

                def scale_g(g, carry2):
                    vals = valv[cidx, pl.ds(g * _LANES, _LANES)]
                    for t in range(_LANES):
                        e = i * 128 + g * _LANES + t
                        bv = vals.at[jnp.full((_LANES,), t, jnp.int32)].get(
                            mode="promise_in_bounds")
                        rbuf[p, e, :] = rbuf[p, e, :] * bv
                    return carry2
                lax.fori_loop(0, 8, scale_g, 0)
                pltpu.async_copy(rbuf.at[p, pl.ds(i * 128, 128)],
                                 acc.at[rowv.at[cidx]], ssem, add=True)
            return carry
        lax.fori_loop(0, n_steps, superstep, 0)
        pltpu.make_async_copy(rbuf.at[(n_steps - 1) % 2],
                              acc.at[pl.ds(0, grp * 128)], ssem).wait()
        plsc.subcore_barrier()

        pltpu.sync_copy(acc.at[pl.ds(tid * rows_per_tile, rows_per_tile)],
                        out_hbm.at[c, pl.ds(tid * rows_per_tile, rows_per_tile)])

    return spmm


def kernel(H, adj_row, adj_col, adj_val, W1, b1, W2, b2):
    n, d = H.shape
    hid = W1.shape[1]
    ncls = W2.shape[1]
    e_tot = adj_row.shape[0]

    n_chunks = -(-e_tot // (_NC * _NS * 128 * _GRP)) * _GRP
    e_pad = _NC * _NS * 128 * n_chunks
    pad = e_pad - e_tot
    pad_idx = jnp.arange(pad, dtype=jnp.int32) % n
    col3 = jnp.concatenate([adj_col, pad_idx]).reshape(_NC * _NS, n_chunks, 128)
    row3 = jnp.concatenate([adj_row, pad_idx]).reshape(_NC * _NS, n_chunks, 128)
    val3 = jnp.concatenate(
        [adj_val, jnp.zeros((pad,), jnp.float32)]).reshape(_NC * _NS, n_chunks, 128)

    spmm1 = _make_spmm(n, hid, n_chunks, idx_shift=3)
    spmm = _make_spmm(n, hid, n_chunks)

    hw1_wide = pl.pallas_call(
        _mm_body,
        out_shape=jax.ShapeDtypeStruct((n, 128), jnp.float32),
    )(H, W1)

    p1 = spmm1(hw1_wide.reshape(8 * n, hid), col3, row3, val3)

    n_pad = p1.shape[1]
    packed_rows = _NC * n_pad * hid // 128
    h1_2d = pl.pallas_call(
        _mid_body,
        out_shape=jax.ShapeDtypeStruct((packed_rows // 2, 128), jnp.float32),
    )(p1.reshape(packed_rows, 128), jnp.tile(b1, 128 // hid).reshape(1, 128))

    p2 = spmm(h1_2d.reshape(n_pad, hid), col3, row3, val3)

    w2big = jnp.kron(jnp.eye(8, dtype=jnp.float32), W2)
    b2big = jnp.tile(b2, 8).reshape(1, 8 * ncls)
    fin_p = pl.pallas_call(
        _fin_body,
        out_shape=jax.ShapeDtypeStruct((packed_rows // 2, 8 * ncls),
                                       jnp.float32),
    )(p2.reshape(packed_rows, 128), w2big, b2big)

    return fin_p.reshape(n_pad, ncls)[:n]


# speedup vs baseline: 1.5460x; 1.0245x over previous
import functools

import jax
import jax.numpy as jnp
from jax import lax
from jax.experimental import pallas as pl
from jax.experimental.pallas import tpu as pltpu
from jax.experimental.pallas import tpu_sc as plsc

_NC = 2
_NS = 16
_LANES = 16
_GRP = 9


def _mm_body(h_ref, w_ref, o_ref):
    hw = jnp.dot(h_ref[...], w_ref[...], preferred_element_type=jnp.float32)
    o_ref[...] = jnp.concatenate(
        [hw, jnp.zeros((hw.shape[0], 128 - hw.shape[1]), jnp.float32)], axis=1)


def _mid_body(p_ref, b_ref, o_ref):
    half = o_ref.shape[0]
    o_ref[...] = jnp.maximum(
        p_ref[:half] + p_ref[half:] + b_ref[...], 0.0)


def _fin_body(p_ref, w_ref, b_ref, sm_ref, bm_ref, o_ref):
    half = p_ref.shape[0] // 2
    s2p = p_ref[:half] + p_ref[half:]
    z = jnp.dot(s2p, w_ref[...], preferred_element_type=jnp.float32)
    z = jnp.maximum(z + b_ref[...], 0.0)
    m = jnp.max(z, axis=1, keepdims=True)
    x = z - m
    e = jnp.exp(x)
    s8 = jnp.dot(e, sm_ref[...], preferred_element_type=jnp.float32)
    lse = jnp.log(s8)
    o_ref[...] = x - jnp.dot(lse, bm_ref[...],
                             preferred_element_type=jnp.float32)


@functools.lru_cache(maxsize=None)
def _make_spmm(n_rows: int, width: int, n_chunks: int, idx_shift: int = 0):
    rpt = (n_rows + _NS - 1) // _NS
    rows_per_tile = (rpt + 7) // 8 * 8
    n_pad = rows_per_tile * _NS
    grp = _GRP
    n_steps = n_chunks // grp
    assert n_steps * grp == n_chunks
    mesh = plsc.VectorSubcoreMesh(core_axis_name="c", subcore_axis_name="s")

    @functools.partial(
        pl.kernel,
        out_type=jax.ShapeDtypeStruct((_NC, n_pad, width), jnp.float32),
        mesh=mesh,
        scratch_types=[
            pltpu.VMEM((n_chunks, 128), jnp.int32),
            pltpu.VMEM((n_chunks, 128), jnp.int32),
            pltpu.VMEM((n_chunks, 128), jnp.float32),
            pltpu.VMEM((2, grp * 128, width), jnp.float32),
            pltpu.VMEM((rows_per_tile, width), jnp.float32),
            pltpu.VMEM_SHARED((n_pad, width), jnp.float32),
            pltpu.SemaphoreType.DMA,
            pltpu.SemaphoreType.DMA,
        ],
        compiler_params=pltpu.CompilerParams(use_tc_tiling_on_sc=False),
    )
    def spmm(mat_hbm, col_hbm, row_hbm, val_hbm, out_hbm,
             colv, rowv, valv, rbuf, zbuf, acc, gsem, ssem):
        c = lax.axis_index("c")
        tid = lax.axis_index("s")
        w = c * _NS + tid

        pltpu.sync_copy(col_hbm.at[w], colv)
        d_row = pltpu.async_copy(row_hbm.at[w], rowv, ssem)
        d_val = pltpu.async_copy(val_hbm.at[w], valv, ssem)

        if idx_shift:
            def shift_body(j, carry):
                for g in range(8):
                    sl = pl.ds(g * _LANES, _LANES)
                    colv[j, sl] = colv[j, sl] * jnp.int32(1 << idx_shift)
                return carry
            lax.fori_loop(0, n_chunks, shift_body, 0)

        for i in range(grp):
            pltpu.async_copy(mat_hbm.at[colv.at[i]],
                             rbuf.at[0, pl.ds(i * 128, 128)], gsem)

        def zbody(i, carry):
            for u in range(4):
                zbuf[i * 4 + u, :] = jnp.zeros((width,), jnp.float32)
            return carry
        lax.fori_loop(0, rows_per_tile // 4, zbody, 0)
        pltpu.sync_copy(zbuf, acc.at[pl.ds(tid * rows_per_tile, rows_per_tile)])
        d_row.wait()
        d_val.wait()
        plsc.subcore_barrier()

        def superstep(s, carry):
            p = lax.rem(s, 2)
            pltpu.make_async_copy(mat_hbm.at[pl.ds(0, grp * 128)],
                                  rbuf.at[p], gsem).wait()

            @pl.when(s > 0)
            def _():
                pltpu.make_async_copy(rbuf.at[1 - p],
                                      acc.at[pl.ds(0, grp * 128)],
                                      ssem).wait()

            @pl.when(s < n_steps - 1)
            def _():
                base = jnp.minimum(s + 1, n_steps - 1) * grp
                for i in range(grp):
                    pltpu.async_copy(mat_hbm.at[colv.at[base + i]],
                                     rbuf.at[1 - p, pl.ds(i * 128, 128)], gsem)

            for i in range(grp):
                cidx = s * grp + i

                def scale_g(g, carry2):
                    vals = valv[cidx, pl.ds(g * _LANES, _LANES)]
                    for t in range(_LANES):
                        e = i * 128 + g * _LANES + t
                        bv = vals.at[jnp.full((_LANES,), t, jnp.int32)].get(
                            mode="promise_in_bounds")
                        rbuf[p, e, :] = rbuf[p, e, :] * bv
                    return carry2
                lax.fori_loop(0, 8, scale_g, 0)
                pltpu.async_copy(rbuf.at[p, pl.ds(i * 128, 128)],
                                 acc.at[rowv.at[cidx]], ssem, add=True)
            return carry
        lax.fori_loop(0, n_steps, superstep, 0)
        pltpu.make_async_copy(rbuf.at[(n_steps - 1) % 2],
                              acc.at[pl.ds(0, grp * 128)], ssem).wait()
        plsc.subcore_barrier()

        pltpu.sync_copy(acc.at[pl.ds(tid * rows_per_tile, rows_per_tile)],
                        out_hbm.at[c, pl.ds(tid * rows_per_tile, rows_per_tile)])

    return spmm


def kernel(H, adj_row, adj_col, adj_val, W1, b1, W2, b2):
    n, d = H.shape
    hid = W1.shape[1]
    ncls = W2.shape[1]
    e_tot = adj_row.shape[0]

    n_chunks = -(-e_tot // (_NC * _NS * 128 * _GRP)) * _GRP
    e_pad = _NC * _NS * 128 * n_chunks
    pad = e_pad - e_tot
    pad_idx = jnp.arange(pad, dtype=jnp.int32) % n
    col3 = jnp.concatenate([adj_col, pad_idx]).reshape(_NC * _NS, n_chunks, 128)
    row3 = jnp.concatenate([adj_row, pad_idx]).reshape(_NC * _NS, n_chunks, 128)
    val3 = jnp.concatenate(
        [adj_val, jnp.zeros((pad,), jnp.float32)]).reshape(_NC * _NS, n_chunks, 128)

    spmm1 = _make_spmm(n, hid, n_chunks, idx_shift=3)
    spmm = _make_spmm(n, hid, n_chunks)

    hw1_wide = pl.pallas_call(
        _mm_body,
        out_shape=jax.ShapeDtypeStruct((n, 128), jnp.float32),
    )(H, W1)

    p1 = spmm1(hw1_wide.reshape(8 * n, hid), col3, row3, val3)

    n_pad = p1.shape[1]
    packed_rows = _NC * n_pad * hid // 128
    h1_2d = pl.pallas_call(
        _mid_body,
        out_shape=jax.ShapeDtypeStruct((packed_rows // 2, 128), jnp.float32),
    )(p1.reshape(packed_rows, 128), jnp.tile(b1, 128 // hid).reshape(1, 128))

    p2 = spmm(h1_2d.reshape(n_pad, hid), col3, row3, val3)

    w2big = jnp.kron(jnp.eye(8, dtype=jnp.float32), W2)
    b2big = jnp.tile(b2, 8).reshape(1, 8 * ncls)
    eye8 = jnp.eye(8, dtype=jnp.float32)
    sum_mat = jnp.kron(eye8, jnp.ones((ncls, 1), jnp.float32))
    bcast_mat = jnp.kron(eye8, jnp.ones((1, ncls), jnp.float32))
    fin_p = pl.pallas_call(
        _fin_body,
        out_shape=jax.ShapeDtypeStruct((packed_rows // 2, 8 * ncls),
                                       jnp.float32),
    )(p2.reshape(packed_rows, 128), w2big, b2big, sum_mat, bcast_mat)

    return fin_p[:n // 8].reshape(n, ncls)
